# baseline (speedup 1.0000x reference)
import functools

import jax
import jax.numpy as jnp
from jax import lax
from jax.experimental import pallas as pl
from jax.experimental.pallas import tpu as pltpu
from jax.experimental.pallas import tpu_sc as plsc

_B, _S, _D, _E, _K = 4, 2048, 4096, 64, 8
_N = _B * _S
_LOAD_BALANCE_SCALE = 0.01
_NOISY_STD = 1.0

_RT = 512
_SC_WORKERS = 32
_RPW = _N // _SC_WORKERS


def _gate_body(x_ref, w_ref, nw_ref, nz_ref, out_ref, loss_ref, acc_ref):
    i = pl.program_id(0)
    logits = lax.dot_general(
        x_ref[...], w_ref[...], (((1,), (1,)), ((), ())),
        preferred_element_type=jnp.float32)

    m = jnp.max(logits, axis=1, keepdims=True)
    e = jnp.exp(logits - m)
    gw = e / jnp.sum(e, axis=1, keepdims=True)

    @pl.when(i == 0)
    def _():
        acc_ref[...] = jnp.zeros_like(acc_ref)

    acc_ref[...] += jnp.sum(gw, axis=0, keepdims=True)
    out_ref[...] = logits + nz_ref[...] * nw_ref[...]

    @pl.when(i == pl.num_programs(0) - 1)
    def _():
        usage = acc_ref[...] / _N
        dev = usage - (1.0 / _E)
        loss_ref[...] = (jnp.sum(dev * dev) / _E * _LOAD_BALANCE_SCALE).reshape(1, 1)


def _gate_logits(x2d, W_g, nw_row, nz2d):
    return pl.pallas_call(
        _gate_body,
        grid=(_N // _RT,),
        in_specs=[
            pl.BlockSpec((_RT, _D), lambda i: (i, 0)),
            pl.BlockSpec((_E, _D), lambda i: (0, 0)),
            pl.BlockSpec((1, _E), lambda i: (0, 0)),
            pl.BlockSpec((_RT, _E), lambda i: (i, 0)),
        ],
        out_specs=[
            pl.BlockSpec((_RT, _E), lambda i: (i, 0)),
            pl.BlockSpec((1, 1), lambda i: (0, 0)),
        ],
        out_shape=[
            jax.ShapeDtypeStruct((_N, _E), jnp.float32),
            jax.ShapeDtypeStruct((1, 1), jnp.float32),
        ],
        scratch_shapes=[pltpu.VMEM((1, _E), jnp.float32)],
    )(x2d, W_g, nw_row, nz2d)


def _topk_sc(logits):
    mesh = plsc.VectorSubcoreMesh(core_axis_name="c", subcore_axis_name="s")

    @functools.partial(
        pl.kernel,
        out_type=[
            jax.ShapeDtypeStruct((_N, _E), jnp.float32),
            jax.ShapeDtypeStruct((_N * _K,), jnp.int32),
        ],
        mesh=mesh,
        scratch_types=[
            pltpu.VMEM((_RPW, _E), jnp.float32),
            pltpu.VMEM((_RPW, _E), jnp.float32),
            pltpu.VMEM((_RPW * _K,), jnp.int32),
        ],
        compiler_params=pltpu.CompilerParams(needs_layout_passes=False),
    )
    def k(logits_hbm, w_hbm, idx_hbm, vals_v, w_v, idx_v):
        wid = lax.axis_index("s") * 2 + lax.axis_index("c")
        base = wid * _RPW
        pltpu.sync_copy(logits_hbm.at[pl.ds(base, _RPW), :], vals_v)

        lanes = lax.iota(jnp.int32, 16)
        lt8 = lanes < 8

        def merge(av, ai, bv, bi):
            mv = jnp.where(lt8, av, lax.rev(bv, (0,)))
            mi = jnp.where(lt8, ai, lax.rev(bi, (0,)))
            return plsc.sort_key_val(mv, mi, descending=True)

        @plsc.parallel_loop(0, _RPW, unroll=4)
        def row_body(r):
            vs = [vals_v[r, pl.ds(j * 16, 16)] for j in range(4)]
            svs, sis = [], []
            for j in range(4):
                sv, si = plsc.sort_key_val(vs[j], lanes + j * 16, descending=True)
                svs.append(sv)
                sis.append(si)
            d01v, d01i = merge(svs[0], sis[0], svs[1], sis[1])
            d23v, d23i = merge(svs[2], sis[2], svs[3], sis[3])
            fv, fi = merge(d01v, d01i, d23v, d23i)

            m = jnp.max(fv)
            t8 = jnp.min(jnp.where(lt8, fv, jnp.inf))
            ex = jnp.exp(fv - m)
            denom = jnp.broadcast_to(jnp.sum(jnp.where(lt8, ex, 0.0)), (16,))
            inv = jnp.ones((16,), jnp.float32) / denom
            for j in range(4):
                wj = jnp.where(vs[j] >= t8, jnp.exp(vs[j] - m) * inv, 0.0)
                w_v[r, pl.ds(j * 16, 16)] = wj
            plsc.store_scatter(idx_v, [r * _K + lanes], fi, mask=lt8)

        pltpu.sync_copy(w_v, w_hbm.at[pl.ds(base, _RPW), :])
        pltpu.sync_copy(idx_v, idx_hbm.at[pl.ds(base * _K, _RPW * _K)])

    return k(logits)


def kernel(x, W_g, noise_weight, noise_raw):
    x2d = x.reshape(_N, _D)
    nz2d = noise_raw.reshape(_N, _E)
    nw_row = (noise_weight * _NOISY_STD).reshape(1, _E)
    logits_noisy, loss = _gate_logits(x2d, W_g, nw_row, nz2d)
    w_flat, idx_flat = _topk_sc(logits_noisy)
    return (
        w_flat.reshape(_B, _S, _E),
        idx_flat.reshape(_B, _S, _K),
        loss.reshape(()),
    )


# speedup vs baseline: 1.0079x; 1.0079x over previous
import functools

import jax
import jax.numpy as jnp
from jax import lax
from jax.experimental import pallas as pl
from jax.experimental.pallas import tpu as pltpu
from jax.experimental.pallas import tpu_sc as plsc

_B, _S, _D, _E, _K = 4, 2048, 4096, 64, 8
_N = _B * _S
_LOAD_BALANCE_SCALE = 0.01
_NOISY_STD = 1.0

_RT = 512
_SC_WORKERS = 32
_RPW = _N // _SC_WORKERS


def _gate_body(x_ref, w_ref, nw_ref, nz_ref, out_ref, loss_ref, acc_ref):
    i = pl.program_id(0)
    logits = lax.dot_general(
        x_ref[...], w_ref[...], (((1,), (1,)), ((), ())),
        preferred_element_type=jnp.float32)

    m = jnp.max(logits, axis=1, keepdims=True)
    e = jnp.exp(logits - m)
    gw = e / jnp.sum(e, axis=1, keepdims=True)

    @pl.when(i == 0)
    def _():
        acc_ref[...] = jnp.zeros_like(acc_ref)

    acc_ref[...] += jnp.sum(gw, axis=0, keepdims=True)
    out_ref[...] = logits + nz_ref[...] * nw_ref[...]

    @pl.when(i == pl.num_programs(0) - 1)
    def _():
        usage = acc_ref[...] / _N
        dev = usage - (1.0 / _E)
        loss_ref[...] = (jnp.sum(dev * dev) / _E * _LOAD_BALANCE_SCALE).reshape(1, 1)


def _gate_logits(x2d, W_g, nw_row, nz2d):
    return pl.pallas_call(
        _gate_body,
        grid=(_N // _RT,),
        in_specs=[
            pl.BlockSpec((_RT, _D), lambda i: (i, 0)),
            pl.BlockSpec((_E, _D), lambda i: (0, 0)),
            pl.BlockSpec((1, _E), lambda i: (0, 0)),
            pl.BlockSpec((_RT, _E), lambda i: (i, 0)),
        ],
        out_specs=[
            pl.BlockSpec((_RT, _E), lambda i: (i, 0)),
            pl.BlockSpec((1, 1), lambda i: (0, 0)),
        ],
        out_shape=[
            jax.ShapeDtypeStruct((_N, _E), jnp.float32),
            jax.ShapeDtypeStruct((1, 1), jnp.float32),
        ],
        scratch_shapes=[pltpu.VMEM((1, _E), jnp.float32)],
    )(x2d, W_g, nw_row, nz2d)


def _topk_sc(logits):
    mesh = plsc.VectorSubcoreMesh(core_axis_name="c", subcore_axis_name="s")

    @functools.partial(
        pl.kernel,
        out_type=[
            jax.ShapeDtypeStruct((_N, _E), jnp.float32),
            jax.ShapeDtypeStruct((_N * _K,), jnp.int32),
        ],
        mesh=mesh,
        scratch_types=[
            pltpu.VMEM((_RPW, _E), jnp.float32),
            pltpu.VMEM((_RPW, _E), jnp.float32),
            pltpu.VMEM((_RPW * _K,), jnp.int32),
        ],
        compiler_params=pltpu.CompilerParams(needs_layout_passes=False),
    )
    def k(logits_hbm, w_hbm, idx_hbm, vals_v, w_v, idx_v):
        wid = lax.axis_index("s") * 2 + lax.axis_index("c")
        base = wid * _RPW
        pltpu.sync_copy(logits_hbm.at[pl.ds(base, _RPW), :], vals_v)

        lanes = lax.iota(jnp.int32, 16)
        lt8 = lanes < 8

        def merge(av, ai, bv, bi):
            mv = jnp.where(lt8, av, lax.rev(bv, (0,)))
            mi = jnp.where(lt8, ai, lax.rev(bi, (0,)))
            return plsc.sort_key_val(mv, mi, descending=True)

        @plsc.parallel_loop(0, _RPW, unroll=2)
        def row_body(r):
            vs = [vals_v[r, pl.ds(j * 16, 16)] for j in range(4)]
            svs, sis = [], []
            for j in range(4):
                sv, si = plsc.sort_key_val(vs[j], lanes + j * 16, descending=True)
                svs.append(sv)
                sis.append(si)
            d01v, d01i = merge(svs[0], sis[0], svs[1], sis[1])
            d23v, d23i = merge(svs[2], sis[2], svs[3], sis[3])
            fv, fi = merge(d01v, d01i, d23v, d23i)

            m = jnp.max(fv)
            t8 = jnp.min(jnp.where(lt8, fv, jnp.inf))
            ex = jnp.exp(fv - m)
            denom = jnp.broadcast_to(jnp.sum(jnp.where(lt8, ex, 0.0)), (16,))
            inv = jnp.ones((16,), jnp.float32) / denom
            for j in range(4):
                wj = jnp.where(vs[j] >= t8, jnp.exp(vs[j] - m) * inv, 0.0)
                w_v[r, pl.ds(j * 16, 16)] = wj
            plsc.store_scatter(idx_v, [r * _K + lanes], fi, mask=lt8)

        pltpu.sync_copy(w_v, w_hbm.at[pl.ds(base, _RPW), :])
        pltpu.sync_copy(idx_v, idx_hbm.at[pl.ds(base * _K, _RPW * _K)])

    return k(logits)


def kernel(x, W_g, noise_weight, noise_raw):
    x2d = x.reshape(_N, _D)
    nz2d = noise_raw.reshape(_N, _E)
    nw_row = (noise_weight * _NOISY_STD).reshape(1, _E)
    logits_noisy, loss = _gate_logits(x2d, W_g, nw_row, nz2d)
    w_flat, idx_flat = _topk_sc(logits_noisy)
    return (
        w_flat.reshape(_B, _S, _E),
        idx_flat.reshape(_B, _S, _K),
        loss.reshape(()),
    )


# speedup vs baseline: 1.0188x; 1.0109x over previous
import functools

import jax
import jax.numpy as jnp
from jax import lax
from jax.experimental import pallas as pl
from jax.experimental.pallas import tpu as pltpu
from jax.experimental.pallas import tpu_sc as plsc

_B, _S, _D, _E, _K = 4, 2048, 4096, 64, 8
_N = _B * _S
_LOAD_BALANCE_SCALE = 0.01
_NOISY_STD = 1.0

_RT = 512
_SC_WORKERS = 32
_RPW = _N // _SC_WORKERS
_NB = 4
_RPB = _RPW // _NB


def _gate_body(x_ref, w_ref, nw_ref, nz_ref, out_ref, loss_ref, acc_ref):
    i = pl.program_id(0)
    logits = lax.dot_general(
        x_ref[...], w_ref[...], (((1,), (1,)), ((), ())),
        preferred_element_type=jnp.float32)

    m = jnp.max(logits, axis=1, keepdims=True)
    e = jnp.exp(logits - m)
    gw = e / jnp.sum(e, axis=1, keepdims=True)

    @pl.when(i == 0)
    def _():
        acc_ref[...] = jnp.zeros_like(acc_ref)

    acc_ref[...] += jnp.sum(gw, axis=0, keepdims=True)
    out_ref[...] = logits + nz_ref[...] * nw_ref[...]

    @pl.when(i == pl.num_programs(0) - 1)
    def _():
        usage = acc_ref[...] / _N
        dev = usage - (1.0 / _E)
        loss_ref[...] = (jnp.sum(dev * dev) / _E * _LOAD_BALANCE_SCALE).reshape(1, 1)


def _gate_logits(x2d, W_g, nw_row, nz2d):
    return pl.pallas_call(
        _gate_body,
        grid=(_N // _RT,),
        in_specs=[
            pl.BlockSpec((_RT, _D), lambda i: (i, 0)),
            pl.BlockSpec((_E, _D), lambda i: (0, 0)),
            pl.BlockSpec((1, _E), lambda i: (0, 0)),
            pl.BlockSpec((_RT, _E), lambda i: (i, 0)),
        ],
        out_specs=[
            pl.BlockSpec((_RT, _E), lambda i: (i, 0)),
            pl.BlockSpec((1, 1), lambda i: (0, 0)),
        ],
        out_shape=[
            jax.ShapeDtypeStruct((_N, _E), jnp.float32),
            jax.ShapeDtypeStruct((1, 1), jnp.float32),
        ],
        scratch_shapes=[pltpu.VMEM((1, _E), jnp.float32)],
    )(x2d, W_g, nw_row, nz2d)


def _topk_sc(logits):
    mesh = plsc.VectorSubcoreMesh(core_axis_name="c", subcore_axis_name="s")

    @functools.partial(
        pl.kernel,
        out_type=[
            jax.ShapeDtypeStruct((_N, _E), jnp.float32),
            jax.ShapeDtypeStruct((_N * _K,), jnp.int32),
        ],
        mesh=mesh,
        scratch_types=[
            pltpu.VMEM((_RPW, _E), jnp.float32),
            pltpu.VMEM((_RPW, _E), jnp.float32),
            pltpu.VMEM((_RPW * _K,), jnp.int32),
            pltpu.SemaphoreType.DMA,
            pltpu.SemaphoreType.DMA,
        ],
        compiler_params=pltpu.CompilerParams(needs_layout_passes=False),
    )
    def k(logits_hbm, w_hbm, idx_hbm, vals_v, w_v, idx_v, sem_in, sem_out):
        wid = lax.axis_index("s") * 2 + lax.axis_index("c")
        base = wid * _RPW

        in_d = [
            pltpu.async_copy(
                logits_hbm.at[pl.ds(base + b * _RPB, _RPB), :],
                vals_v.at[pl.ds(b * _RPB, _RPB), :], sem_in)
            for b in range(_NB)
        ]

        lanes = lax.iota(jnp.int32, 16)
        lt8 = lanes < 8

        def merge(av, ai, bv, bi):
            mv = jnp.where(lt8, av, lax.rev(bv, (0,)))
            mi = jnp.where(lt8, ai, lax.rev(bi, (0,)))
            return plsc.sort_key_val(mv, mi, descending=True)

        out_d = []
        for b in range(_NB):
            in_d[b].wait()

            @plsc.parallel_loop(b * _RPB, (b + 1) * _RPB, unroll=2)
            def row_body(r):
                vs = [vals_v[r, pl.ds(j * 16, 16)] for j in range(4)]
                svs, sis = [], []
                for j in range(4):
                    sv, si = plsc.sort_key_val(vs[j], lanes + j * 16, descending=True)
                    svs.append(sv)
                    sis.append(si)
                d01v, d01i = merge(svs[0], sis[0], svs[1], sis[1])
                d23v, d23i = merge(svs[2], sis[2], svs[3], sis[3])
                fv, fi = merge(d01v, d01i, d23v, d23i)

                m = jnp.max(fv)
                t8 = jnp.min(jnp.where(lt8, fv, jnp.inf))
                ex = jnp.exp(fv - m)
                denom = jnp.broadcast_to(jnp.sum(jnp.where(lt8, ex, 0.0)), (16,))
                inv = jnp.ones((16,), jnp.float32) / denom
                for j in range(4):
                    wj = jnp.where(vs[j] >= t8, jnp.exp(vs[j] - m) * inv, 0.0)
                    w_v[r, pl.ds(j * 16, 16)] = wj
                plsc.store_scatter(idx_v, [r * _K + lanes], fi, mask=lt8)

            out_d.append(pltpu.async_copy(
                w_v.at[pl.ds(b * _RPB, _RPB), :],
                w_hbm.at[pl.ds(base + b * _RPB, _RPB), :], sem_out))
            out_d.append(pltpu.async_copy(
                idx_v.at[pl.ds(b * _RPB * _K, _RPB * _K)],
                idx_hbm.at[pl.ds((base + b * _RPB) * _K, _RPB * _K)], sem_out))
        for d in out_d:
            d.wait()

    return k(logits)


def kernel(x, W_g, noise_weight, noise_raw):
    x2d = x.reshape(_N, _D)
    nz2d = noise_raw.reshape(_N, _E)
    nw_row = (noise_weight * _NOISY_STD).reshape(1, _E)
    logits_noisy, loss = _gate_logits(x2d, W_g, nw_row, nz2d)
    w_flat, idx_flat = _topk_sc(logits_noisy)
    return (
        w_flat.reshape(_B, _S, _E),
        idx_flat.reshape(_B, _S, _K),
        loss.reshape(()),
    )
